# baseline (speedup 1.0000x reference)
import functools

import jax
import jax.numpy as jnp
from jax import lax
from jax.experimental import pallas as pl
from jax.experimental.pallas import tpu as pltpu
from jax.experimental.pallas import tpu_sc as plsc

NC, NS, L = 2, 16, 16
NW = NC * NS
E = 320000
EB = 128
NBLK = E // EB
NFULL = NBLK // NW
NEXTRA = NBLK - NFULL * NW
N_PAD = 10240
SCN = 1600
NCH = E // SCN
NEG = -3.0e38

_mesh = plsc.VectorSubcoreMesh(core_axis_name="c", subcore_axis_name="s",
                               num_cores=NC, num_subcores=NS)


def _wid():
    return lax.axis_index("s") * NC + lax.axis_index("c")


def _nblocks(w):
    return NFULL + jnp.where(w < NEXTRA, 1, 0).astype(jnp.int32)


def _splat(v):
    return jnp.full((L,), v, jnp.int32)


def _tc0a_body(x_ref, w_ref, o_ref):
    o_ref[...] = jnp.dot(x_ref[...], w_ref[...], preferred_element_type=jnp.float32)


def _tc0b_body(x_ref, wr_ref, wq_ref, bqr_ref, ar_ref, aq_ref, nr_ref):
    x = x_ref[...]
    ar_ref[...] = jnp.dot(x, wr_ref[...], preferred_element_type=jnp.float32)
    aq_ref[...] = jnp.dot(x, wq_ref[...], preferred_element_type=jnp.float32) + bqr_ref[0, :]
    nr_ref[...] = -x


def _tc1_body(t_ref, u_ref, wa_ref, ba_ref, m_ref):
    t = jnp.maximum(t_ref[...], 0.0)
    a = jnp.dot(t, wa_ref[...], preferred_element_type=jnp.float32) + ba_ref[0, 0]
    alpha = jax.nn.sigmoid(a)
    m = alpha * u_ref[...]
    blk = m.shape[0]
    m_ref[...] = jnp.reshape(
        jnp.transpose(jnp.reshape(m, (blk, 16, 8)), (1, 0, 2)), (16, blk * 8))


def _tc2_body(p_ref, wh_ref, w1_ref, w2_ref, bb_ref, wn_ref, wnb_ref,
              s1_ref, s2_ref, h2_ref):
    p = p_ref[...]
    a = jnp.maximum(p[0], p[1])
    a = jnp.where(a > -1.0e37, a, 0.0)
    agg = jnp.transpose(jnp.reshape(a, (128, a.shape[2])))
    hn = jnp.dot(agg, wh_ref[...], preferred_element_type=jnp.float32)
    s1 = jnp.dot(hn, w1_ref[...], preferred_element_type=jnp.float32) + bb_ref[0, 0]
    s2 = jnp.dot(hn, w2_ref[...], preferred_element_type=jnp.float32)
    s1_ref[...] = jnp.reshape(s1, (s1.shape[0],))
    s2_ref[...] = jnp.reshape(s2, (s2.shape[0],))
    h2_ref[...] = jnp.dot(hn, wn_ref[...], preferred_element_type=jnp.float32) + wnb_ref[0, :]


def _tc3_body(p_ref, o_ref):
    o_ref[...] = p_ref[0] + p_ref[1]


def _tc0a(hidden, ws):
    blk = 2000
    return pl.pallas_call(
        _tc0a_body,
        grid=(hidden.shape[0] // blk,),
        in_specs=[pl.BlockSpec((blk, 128), lambda i: (i, 0)),
                  pl.BlockSpec((128, 128), lambda i: (0, 0))],
        out_specs=pl.BlockSpec((blk, 128), lambda i: (i, 0)),
        out_shape=jax.ShapeDtypeStruct((hidden.shape[0], 128), jnp.float32),
    )(hidden, ws)


def _tc0b(rp, wr, wq, bqr):
    blk = 2048
    return pl.pallas_call(
        _tc0b_body,
        grid=(N_PAD // blk,),
        in_specs=[pl.BlockSpec((blk, 128), lambda i: (i, 0)),
                  pl.BlockSpec((128, 128), lambda i: (0, 0)),
                  pl.BlockSpec((128, 128), lambda i: (0, 0)),
                  pl.BlockSpec((1, 128), lambda i: (0, 0))],
        out_specs=[pl.BlockSpec((blk, 128), lambda i: (i, 0))] * 3,
        out_shape=[jax.ShapeDtypeStruct((N_PAD, 128), jnp.float32)] * 3,
    )(rp, wr, wq, bqr.reshape(1, 128))


def _tc1(T, U, w_alpha, b_alpha):
    blk = 2560
    return pl.pallas_call(
        _tc1_body,
        grid=(E // blk,),
        in_specs=[pl.BlockSpec((blk, 128), lambda i: (i, 0)),
                  pl.BlockSpec((blk, 128), lambda i: (i, 0)),
                  pl.BlockSpec((128, 1), lambda i: (0, 0)),
                  pl.BlockSpec((1, 1), lambda i: (0, 0))],
        out_specs=pl.BlockSpec((16, blk * 8), lambda i: (0, i)),
        out_shape=jax.ShapeDtypeStruct((16, E * 8), jnp.float32),
    )(T, U, w_alpha, b_alpha.reshape(1, 1))


def _tc2(agg, wh, w1, w2, bb, wn, wnb):
    blk = 2048
    return pl.pallas_call(
        _tc2_body,
        grid=(N_PAD // blk,),
        in_specs=[pl.BlockSpec((2, 16, 8, blk), lambda i: (0, 0, 0, i)),
                  pl.BlockSpec((128, 128), lambda i: (0, 0)),
                  pl.BlockSpec((128, 1), lambda i: (0, 0)),
                  pl.BlockSpec((128, 1), lambda i: (0, 0)),
                  pl.BlockSpec((1, 1), lambda i: (0, 0)),
                  pl.BlockSpec((128, 128), lambda i: (0, 0)),
                  pl.BlockSpec((1, 128), lambda i: (0, 0))],
        out_specs=[pl.BlockSpec((blk,), lambda i: (i,)),
                   pl.BlockSpec((blk,), lambda i: (i,)),
                   pl.BlockSpec((blk, 128), lambda i: (i, 0))],
        out_shape=[jax.ShapeDtypeStruct((N_PAD,), jnp.float32),
                   jax.ShapeDtypeStruct((N_PAD,), jnp.float32),
                   jax.ShapeDtypeStruct((N_PAD, 128), jnp.float32)],
    )(agg, wh, w1, w2, bb.reshape(1, 1), wn, wnb.reshape(1, 128))


def _tc3(parts):
    blk = 2048
    return pl.pallas_call(
        _tc3_body,
        grid=(N_PAD // blk,),
        in_specs=[pl.BlockSpec((2, blk, 128), lambda i: (0, i, 0))],
        out_specs=pl.BlockSpec((blk, 128), lambda i: (i, 0)),
        out_shape=jax.ShapeDtypeStruct((N_PAD, 128), jnp.float32),
    )(parts)


def _sc_a_body(edges, qrel, a_s, a_r, a_q, hid, nrel,
               t_out, u_out, subc, objc,
               edg_v, qrl_v, sub_v, rel_v, q_v, obj_v, rs_v, rr_v, rq_v):
    w = _wid()
    iot = lax.iota(jnp.int32, L)
    pltpu.sync_copy(qrel, qrl_v)
    nb = _nblocks(w)

    def blk_body(j, carry):
        b = w + j * NW
        base = b * EB
        pltpu.sync_copy(edges.at[pl.ds(base * 6, EB * 6)], edg_v)

        def grp(g, c):
            rows = (g * L + iot) * 6
            c0 = plsc.load_gather(edg_v, [rows])
            c2 = plsc.load_gather(edg_v, [rows + 2])
            c4 = plsc.load_gather(edg_v, [rows + 4])
            c5 = plsc.load_gather(edg_v, [rows + 5])
            q = plsc.load_gather(qrl_v, [c0])
            sub_v[pl.ds(g * L, L)] = c4
            rel_v[pl.ds(g * L, L)] = c2
            q_v[pl.ds(g * L, L)] = q
            obj_v[pl.ds(g * L, L)] = c5
            return c

        lax.fori_loop(0, EB // L, grp, 0)
        pltpu.sync_copy(a_s.at[sub_v], rs_v)
        pltpu.sync_copy(a_r.at[rel_v], rr_v)
        pltpu.sync_copy(a_q.at[q_v], rq_v)

        def addt(e, c):
            for k in range(8):
                sl = pl.ds(k * L, L)
                rs_v[e, sl] = rs_v[e, sl] + rr_v[e, sl] + rq_v[e, sl]
            return c

        lax.fori_loop(0, EB, addt, 0)
        pltpu.sync_copy(rs_v, t_out.at[pl.ds(base, EB), :])
        pltpu.sync_copy(hid.at[sub_v], rr_v)
        pltpu.sync_copy(nrel.at[rel_v], rq_v)

        def addu(e, c):
            for k in range(8):
                sl = pl.ds(k * L, L)
                rr_v[e, sl] = rr_v[e, sl] + rq_v[e, sl]
            return c

        lax.fori_loop(0, EB, addu, 0)
        pltpu.sync_copy(rr_v, u_out.at[pl.ds(base, EB), :])
        pltpu.sync_copy(sub_v, subc.at[pl.ds(base, EB)])
        pltpu.sync_copy(obj_v, objc.at[pl.ds(base, EB)])
        return carry

    lax.fori_loop(0, nb, blk_body, 0)


def _sc_a(edges, q_rel, a_s, a_r, a_q, hidden, nrel):
    f = pl.kernel(
        _sc_a_body,
        out_type=[jax.ShapeDtypeStruct((E, 128), jnp.float32),
                  jax.ShapeDtypeStruct((E, 128), jnp.float32),
                  jax.ShapeDtypeStruct((E,), jnp.int32),
                  jax.ShapeDtypeStruct((E,), jnp.int32)],
        mesh=_mesh,
        compiler_params=pltpu.CompilerParams(needs_layout_passes=False),
        scratch_types=[pltpu.VMEM((EB * 6,), jnp.int32),
                       pltpu.VMEM((10000,), jnp.int32),
                       pltpu.VMEM((EB,), jnp.int32),
                       pltpu.VMEM((EB,), jnp.int32),
                       pltpu.VMEM((EB,), jnp.int32),
                       pltpu.VMEM((EB,), jnp.int32),
                       pltpu.VMEM((EB, 128), jnp.float32),
                       pltpu.VMEM((EB, 128), jnp.float32),
                       pltpu.VMEM((EB, 128), jnp.float32)],
    )
    return f(edges, q_rel, a_s, a_r, a_q, hidden, nrel)


CH2 = 1280
EHALF = E // NC
NCH2 = EHALF // CH2


def _sc_b_body(objc, m_in, part_out, obj_v, ms_v, agg_v):
    h = lax.axis_index("c")
    s = lax.axis_index("s")
    iot = lax.iota(jnp.int32, L)
    lane_f = iot & 7
    half = lax.shift_right_logical(iot, 3)
    negv = jnp.full((L,), NEG, jnp.float32)

    def ini(i, c):
        plsc.store_scatter(agg_v, [lane_f, i * 2 + half], negv)
        return c

    lax.fori_loop(0, N_PAD // 2, ini, 0)

    def chunk(c, carry):
        ebase = h * EHALF + c * CH2
        pltpu.sync_copy(objc.at[pl.ds(ebase, CH2)], obj_v)
        pltpu.sync_copy(m_in.at[s, pl.ds(ebase * 8, CH2 * 8)], ms_v)

        def vec(g, cc):
            erow = g * 2 + half
            orep = plsc.load_gather(obj_v, [erow])
            mv = ms_v[pl.ds(g * L, L)]
            rot8 = (iot + 8) & 15
            orot = orep.at[rot8].get(mode="promise_in_bounds")
            mrot = mv.at[rot8].get(mode="promise_in_bounds")
            mv = jnp.where(orep == orot, jnp.maximum(mv, mrot), mv)
            av = plsc.load_gather(agg_v, [lane_f, orep])
            plsc.store_scatter(agg_v, [lane_f, orep], jnp.maximum(av, mv))
            return cc

        lax.fori_loop(0, CH2 // 2, vec, 0)
        return carry

    lax.fori_loop(0, NCH2, chunk, 0)
    pltpu.sync_copy(agg_v, part_out.at[h, s])


def _sc_b(objc, m_in):
    f = pl.kernel(
        _sc_b_body,
        out_type=jax.ShapeDtypeStruct((NC, NS, 8, N_PAD), jnp.float32),
        mesh=_mesh,
        compiler_params=pltpu.CompilerParams(needs_layout_passes=False),
        scratch_types=[pltpu.VMEM((CH2,), jnp.int32),
                       pltpu.VMEM((CH2 * 8,), jnp.float32),
                       pltpu.VMEM((8, N_PAD), jnp.float32)],
    )
    return f(objc, m_in)


def _sc_c1_body(subc, objc, s1, s2, scores_out, stats_out,
                sub_v, obj_v, s1_v, s2_v, sc_v, st_v):
    w = _wid()
    iot = lax.iota(jnp.int32, L)
    nb = _nblocks(w)
    m0 = jnp.full((L,), NEG, jnp.float32)
    s0 = jnp.zeros((L,), jnp.float32)

    def blk(j, MS):
        b = w + j * NW
        base = b * EB
        pltpu.sync_copy(subc.at[pl.ds(base, EB)], sub_v)
        pltpu.sync_copy(objc.at[pl.ds(base, EB)], obj_v)
        pltpu.sync_copy(s1.at[sub_v], s1_v)
        pltpu.sync_copy(s2.at[obj_v], s2_v)

        def grp(g, MS2):
            M, S = MS2
            x = s1_v[pl.ds(g * L, L)] + s2_v[pl.ds(g * L, L)]
            x = jnp.maximum(x, 0.2 * x)
            sc_v[pl.ds(g * L, L)] = x
            mn = jnp.maximum(M, x)
            S = S * jnp.exp(M - mn) + jnp.exp(x - mn)
            return (mn, S)

        MS = lax.fori_loop(0, EB // L, grp, MS)
        pltpu.sync_copy(sc_v, scores_out.at[pl.ds(base, EB)])
        return MS

    M, S = lax.fori_loop(0, nb, blk, (m0, s0))
    plsc.store_scatter(st_v, [_splat(0), iot], M)
    plsc.store_scatter(st_v, [_splat(1), iot], S)
    pltpu.sync_copy(st_v, stats_out.at[w])


def _sc_c1(subc, objc, s1, s2):
    f = pl.kernel(
        _sc_c1_body,
        out_type=[jax.ShapeDtypeStruct((E,), jnp.float32),
                  jax.ShapeDtypeStruct((NW, 2, L), jnp.float32)],
        mesh=_mesh,
        compiler_params=pltpu.CompilerParams(needs_layout_passes=False),
        scratch_types=[pltpu.VMEM((EB,), jnp.int32),
                       pltpu.VMEM((EB,), jnp.int32),
                       pltpu.VMEM((EB,), jnp.float32),
                       pltpu.VMEM((EB,), jnp.float32),
                       pltpu.VMEM((EB,), jnp.float32),
                       pltpu.VMEM((2, L), jnp.float32)],
    )
    return f(subc, objc, s1, s2)


def _sc_c2_body(subc, objc, scores, stats, h2, out_p,
                sub_v, obj_v, sc_v, w_v, h2r_v, z_v, st_v, osh):
    w = _wid()
    sid = lax.axis_index("s")
    cid = lax.axis_index("c")
    iot = lax.iota(jnp.int32, L)
    pltpu.sync_copy(stats, st_v)

    def comb(t, MS):
        M, S = MS
        mv = plsc.load_gather(st_v, [_splat(t), _splat(0), iot])
        sv = plsc.load_gather(st_v, [_splat(t), _splat(1), iot])
        mn = jnp.maximum(M, mv)
        S = S * jnp.exp(M - mn) + sv * jnp.exp(mv - mn)
        return (mn, S)

    M, S = lax.fori_loop(0, NW, comb,
                         (jnp.full((L,), NEG, jnp.float32),
                          jnp.zeros((L,), jnp.float32)))
    ms = lax.reduce_max(M, (0,))
    zs = lax.reduce_sum(S * jnp.exp(M - ms), (0,))
    inv = jnp.ones((L,), jnp.float32) / jnp.broadcast_to(zs, (L,))

    def z(i, c):
        for k in range(8):
            z_v[i, pl.ds(k * L, L)] = jnp.zeros((L,), jnp.float32)
        return c

    lax.fori_loop(0, EB, z, 0)
    for r in range(5):
        pltpu.sync_copy(z_v, osh.at[pl.ds(sid * 640 + r * EB, EB), :])
    plsc.subcore_barrier()

    nb = _nblocks(w)

    def blk(j, carry):
        b = w + j * NW
        base = b * EB
        pltpu.sync_copy(subc.at[pl.ds(base, EB)], sub_v)
        pltpu.sync_copy(objc.at[pl.ds(base, EB)], obj_v)
        pltpu.sync_copy(scores.at[pl.ds(base, EB)], sc_v)
        pltpu.sync_copy(h2.at[obj_v], h2r_v)

        def grp(g, c):
            x = sc_v[pl.ds(g * L, L)]
            w_v[pl.ds(g * L, L)] = jnp.exp(x - ms) * inv
            return c

        lax.fori_loop(0, EB // L, grp, 0)

        def scale(i, c):
            wv = plsc.load_gather(w_v, [_splat(i)])
            for k in range(8):
                sl = pl.ds(k * L, L)
                h2r_v[i, sl] = h2r_v[i, sl] * wv
            return c

        lax.fori_loop(0, EB, scale, 0)
        pltpu.sync_copy(h2r_v, osh.at[sub_v], add=True)
        return carry

    lax.fori_loop(0, nb, blk, 0)
    plsc.subcore_barrier()
    for r in range(5):
        pltpu.sync_copy(osh.at[pl.ds(sid * 640 + r * EB, EB), :],
                        out_p.at[cid, pl.ds(sid * 640 + r * EB, EB), :])


def _sc_c2(subc, objc, scores, stats, h2):
    f = pl.kernel(
        _sc_c2_body,
        out_type=jax.ShapeDtypeStruct((NC, N_PAD, 128), jnp.float32),
        mesh=_mesh,
        compiler_params=pltpu.CompilerParams(needs_layout_passes=False),
        scratch_types=[pltpu.VMEM((EB,), jnp.int32),
                       pltpu.VMEM((EB,), jnp.int32),
                       pltpu.VMEM((EB,), jnp.float32),
                       pltpu.VMEM((EB,), jnp.float32),
                       pltpu.VMEM((EB, 128), jnp.float32),
                       pltpu.VMEM((EB, 128), jnp.float32),
                       pltpu.VMEM((NW, 2, L), jnp.float32),
                       pltpu.VMEM_SHARED((N_PAD, 128), jnp.float32)],
    )
    return f(subc, objc, scores, stats, h2)


def kernel(q_sub, q_rel, hidden, edges, nodes, old_nodes_new_idx, batchsize,
           rela_embed, Ws_attn, Wr_attn, Wqr_attn, bqr, w_alpha, b_alpha, W_h,
           attn_fc_w, attn_fc_b, W_node_w, W_node_b):
    rp = jnp.pad(rela_embed, ((0, N_PAD - rela_embed.shape[0]), (0, 0)))
    edges = edges.astype(jnp.int32)
    q_rel = q_rel.astype(jnp.int32)

    a_s = _tc0a(hidden, Ws_attn)
    a_r, a_q, nrel = _tc0b(rp, Wr_attn, Wqr_attn, bqr)
    T, U, subc, objc = _sc_a(edges.reshape(-1), q_rel, a_s, a_r, a_q, hidden, nrel)
    M = _tc1(T, U, w_alpha, b_alpha)
    agg = _sc_b(objc, M)
    s1, s2, h2 = _tc2(agg, W_h, attn_fc_w[:128], attn_fc_w[128:],
                      attn_fc_b, W_node_w, W_node_b)
    scores, stats = _sc_c1(subc, objc, s1, s2)
    parts = _sc_c2(subc, objc, scores, stats, h2)
    out = _tc3(parts)
    return out[:10000]


# speedup vs baseline: 1.3879x; 1.3879x over previous
import functools

import jax
import jax.numpy as jnp
from jax import lax
from jax.experimental import pallas as pl
from jax.experimental.pallas import tpu as pltpu
from jax.experimental.pallas import tpu_sc as plsc

NC, NS, L = 2, 16, 16
NW = NC * NS
E = 320000
EB = 128
NBLK = E // EB
NFULL = NBLK // NW
NEXTRA = NBLK - NFULL * NW
N_PAD = 10240
SCN = 1600
NCH = E // SCN
NEG = -3.0e38

_mesh = plsc.VectorSubcoreMesh(core_axis_name="c", subcore_axis_name="s",
                               num_cores=NC, num_subcores=NS)


def _wid():
    return lax.axis_index("s") * NC + lax.axis_index("c")


def _nblocks(w):
    return NFULL + jnp.where(w < NEXTRA, 1, 0).astype(jnp.int32)


def _splat(v):
    return jnp.full((L,), v, jnp.int32)


def _tc0a_body(x_ref, w_ref, o_ref):
    o_ref[...] = jnp.dot(x_ref[...], w_ref[...], preferred_element_type=jnp.float32)


def _tc0b_body(x_ref, wr_ref, wq_ref, bqr_ref, ar_ref, aq_ref, nr_ref):
    x = x_ref[...]
    ar_ref[...] = jnp.dot(x, wr_ref[...], preferred_element_type=jnp.float32)
    aq_ref[...] = jnp.dot(x, wq_ref[...], preferred_element_type=jnp.float32) + bqr_ref[0, :]
    nr_ref[...] = -x


def _tc1_body(t_ref, u_ref, wa_ref, ba_ref, m_ref):
    t = jnp.maximum(t_ref[...], 0.0)
    a = jnp.dot(t, wa_ref[...], preferred_element_type=jnp.float32) + ba_ref[0, 0]
    alpha = jax.nn.sigmoid(a)
    m_ref[...] = jnp.transpose(alpha * u_ref[...])


def _tc2_body(p_ref, wh_ref, w1_ref, w2_ref, bb_ref, wn_ref, wnb_ref,
              s1_ref, s2_ref, h2_ref):
    p = p_ref[...]
    a = jnp.maximum(p[0], p[1])
    a = jnp.where(a > -1.0e37, a, 0.0)
    agg = jnp.transpose(jnp.reshape(a, (128, a.shape[2])))
    hn = jnp.dot(agg, wh_ref[...], preferred_element_type=jnp.float32)
    s1 = jnp.dot(hn, w1_ref[...], preferred_element_type=jnp.float32) + bb_ref[0, 0]
    s2 = jnp.dot(hn, w2_ref[...], preferred_element_type=jnp.float32)
    s1_ref[...] = jnp.reshape(s1, (s1.shape[0],))
    s2_ref[...] = jnp.reshape(s2, (s2.shape[0],))
    h2_ref[...] = jnp.dot(hn, wn_ref[...], preferred_element_type=jnp.float32) + wnb_ref[0, :]


def _tc3_body(p_ref, o_ref):
    o_ref[...] = p_ref[0] + p_ref[1]


def _tc0a(hidden, ws):
    blk = 2000
    return pl.pallas_call(
        _tc0a_body,
        grid=(hidden.shape[0] // blk,),
        in_specs=[pl.BlockSpec((blk, 128), lambda i: (i, 0)),
                  pl.BlockSpec((128, 128), lambda i: (0, 0))],
        out_specs=pl.BlockSpec((blk, 128), lambda i: (i, 0)),
        out_shape=jax.ShapeDtypeStruct((hidden.shape[0], 128), jnp.float32),
    )(hidden, ws)


def _tc0b(rp, wr, wq, bqr):
    blk = 2048
    return pl.pallas_call(
        _tc0b_body,
        grid=(N_PAD // blk,),
        in_specs=[pl.BlockSpec((blk, 128), lambda i: (i, 0)),
                  pl.BlockSpec((128, 128), lambda i: (0, 0)),
                  pl.BlockSpec((128, 128), lambda i: (0, 0)),
                  pl.BlockSpec((1, 128), lambda i: (0, 0))],
        out_specs=[pl.BlockSpec((blk, 128), lambda i: (i, 0))] * 3,
        out_shape=[jax.ShapeDtypeStruct((N_PAD, 128), jnp.float32)] * 3,
    )(rp, wr, wq, bqr.reshape(1, 128))


def _tc1(T, U, w_alpha, b_alpha):
    blk = 2560
    return pl.pallas_call(
        _tc1_body,
        grid=(E // blk,),
        in_specs=[pl.BlockSpec((blk, 128), lambda i: (i, 0)),
                  pl.BlockSpec((blk, 128), lambda i: (i, 0)),
                  pl.BlockSpec((128, 1), lambda i: (0, 0)),
                  pl.BlockSpec((1, 1), lambda i: (0, 0))],
        out_specs=pl.BlockSpec((128, blk), lambda i: (0, i)),
        out_shape=jax.ShapeDtypeStruct((128, E), jnp.float32),
    )(T, U, w_alpha, b_alpha.reshape(1, 1))


def _tc2(agg, wh, w1, w2, bb, wn, wnb):
    blk = 2048
    return pl.pallas_call(
        _tc2_body,
        grid=(N_PAD // blk,),
        in_specs=[pl.BlockSpec((2, 16, 8, blk), lambda i: (0, 0, 0, i)),
                  pl.BlockSpec((128, 128), lambda i: (0, 0)),
                  pl.BlockSpec((128, 1), lambda i: (0, 0)),
                  pl.BlockSpec((128, 1), lambda i: (0, 0)),
                  pl.BlockSpec((1, 1), lambda i: (0, 0)),
                  pl.BlockSpec((128, 128), lambda i: (0, 0)),
                  pl.BlockSpec((1, 128), lambda i: (0, 0))],
        out_specs=[pl.BlockSpec((blk,), lambda i: (i,)),
                   pl.BlockSpec((blk,), lambda i: (i,)),
                   pl.BlockSpec((blk, 128), lambda i: (i, 0))],
        out_shape=[jax.ShapeDtypeStruct((N_PAD,), jnp.float32),
                   jax.ShapeDtypeStruct((N_PAD,), jnp.float32),
                   jax.ShapeDtypeStruct((N_PAD, 128), jnp.float32)],
    )(agg, wh, w1, w2, bb.reshape(1, 1), wn, wnb.reshape(1, 128))


def _tc3(parts):
    blk = 2048
    return pl.pallas_call(
        _tc3_body,
        grid=(N_PAD // blk,),
        in_specs=[pl.BlockSpec((2, blk, 128), lambda i: (0, i, 0))],
        out_specs=pl.BlockSpec((blk, 128), lambda i: (i, 0)),
        out_shape=jax.ShapeDtypeStruct((N_PAD, 128), jnp.float32),
    )(parts)


def _sc_a_body(edges, qrel, a_s, a_r, a_q, hid, nrel,
               t_out, u_out, subc, objc,
               edg_v, qrl_v, sub_v, rel_v, q_v, obj_v, rs_v, rr_v, rq_v):
    w = _wid()
    iot = lax.iota(jnp.int32, L)
    pltpu.sync_copy(qrel, qrl_v)
    nb = _nblocks(w)

    def blk_body(j, carry):
        b = w + j * NW
        base = b * EB
        pltpu.sync_copy(edges.at[pl.ds(base * 6, EB * 6)], edg_v)

        def grp(g, c):
            rows = (g * L + iot) * 6
            c0 = plsc.load_gather(edg_v, [rows])
            c2 = plsc.load_gather(edg_v, [rows + 2])
            c4 = plsc.load_gather(edg_v, [rows + 4])
            c5 = plsc.load_gather(edg_v, [rows + 5])
            q = plsc.load_gather(qrl_v, [c0])
            sub_v[pl.ds(g * L, L)] = c4
            rel_v[pl.ds(g * L, L)] = c2
            q_v[pl.ds(g * L, L)] = q
            obj_v[pl.ds(g * L, L)] = c5
            return c

        lax.fori_loop(0, EB // L, grp, 0)
        pltpu.sync_copy(a_s.at[sub_v], rs_v)
        pltpu.sync_copy(a_r.at[rel_v], rr_v)
        pltpu.sync_copy(a_q.at[q_v], rq_v)

        def addt(e, c):
            for k in range(8):
                sl = pl.ds(k * L, L)
                rs_v[e, sl] = rs_v[e, sl] + rr_v[e, sl] + rq_v[e, sl]
            return c

        lax.fori_loop(0, EB, addt, 0)
        pltpu.sync_copy(rs_v, t_out.at[pl.ds(base, EB), :])
        pltpu.sync_copy(hid.at[sub_v], rr_v)
        pltpu.sync_copy(nrel.at[rel_v], rq_v)

        def addu(e, c):
            for k in range(8):
                sl = pl.ds(k * L, L)
                rr_v[e, sl] = rr_v[e, sl] + rq_v[e, sl]
            return c

        lax.fori_loop(0, EB, addu, 0)
        pltpu.sync_copy(rr_v, u_out.at[pl.ds(base, EB), :])
        pltpu.sync_copy(sub_v, subc.at[pl.ds(base, EB)])
        pltpu.sync_copy(obj_v, objc.at[pl.ds(base, EB)])
        return carry

    lax.fori_loop(0, nb, blk_body, 0)


def _sc_a(edges, q_rel, a_s, a_r, a_q, hidden, nrel):
    f = pl.kernel(
        _sc_a_body,
        out_type=[jax.ShapeDtypeStruct((E, 128), jnp.float32),
                  jax.ShapeDtypeStruct((E, 128), jnp.float32),
                  jax.ShapeDtypeStruct((E,), jnp.int32),
                  jax.ShapeDtypeStruct((E,), jnp.int32)],
        mesh=_mesh,
        compiler_params=pltpu.CompilerParams(needs_layout_passes=False),
        scratch_types=[pltpu.VMEM((EB * 6,), jnp.int32),
                       pltpu.VMEM((10000,), jnp.int32),
                       pltpu.VMEM((EB,), jnp.int32),
                       pltpu.VMEM((EB,), jnp.int32),
                       pltpu.VMEM((EB,), jnp.int32),
                       pltpu.VMEM((EB,), jnp.int32),
                       pltpu.VMEM((EB, 128), jnp.float32),
                       pltpu.VMEM((EB, 128), jnp.float32),
                       pltpu.VMEM((EB, 128), jnp.float32)],
    )
    return f(edges, q_rel, a_s, a_r, a_q, hidden, nrel)


CH2 = 1280
EHALF = E // NC
NCH2 = EHALF // CH2


def _sc_b_body(objc, m_in, part_out, obj_v, ms_v, agg_v):
    h = lax.axis_index("c")
    s = lax.axis_index("s")
    f0 = s * 8
    iot = lax.iota(jnp.int32, L)
    lane_f = iot & 7
    half = lax.shift_right_logical(iot, 3)
    negv = jnp.full((L,), NEG, jnp.float32)

    def ini(i, c):
        plsc.store_scatter(agg_v, [lane_f, i * 2 + half], negv)
        return c

    lax.fori_loop(0, N_PAD // 2, ini, 0)

    def chunk(c, carry):
        ebase = h * EHALF + c * CH2
        pltpu.sync_copy(objc.at[pl.ds(ebase, CH2)], obj_v)
        pltpu.sync_copy(m_in.at[pl.ds(f0, 8), pl.ds(ebase, CH2)], ms_v)

        def vec(g, cc):
            erow = g * 2 + half
            orep = plsc.load_gather(obj_v, [erow])
            mv = plsc.load_gather(ms_v, [lane_f, erow])
            rot8 = (iot + 8) & 15
            orot = orep.at[rot8].get(mode="promise_in_bounds")
            mrot = mv.at[rot8].get(mode="promise_in_bounds")
            mv = jnp.where(orep == orot, jnp.maximum(mv, mrot), mv)
            av = plsc.load_gather(agg_v, [lane_f, orep])
            plsc.store_scatter(agg_v, [lane_f, orep], jnp.maximum(av, mv))
            return cc

        lax.fori_loop(0, CH2 // 2, vec, 0)
        return carry

    lax.fori_loop(0, NCH2, chunk, 0)
    pltpu.sync_copy(agg_v, part_out.at[h, s])


def _sc_b(objc, m_in):
    f = pl.kernel(
        _sc_b_body,
        out_type=jax.ShapeDtypeStruct((NC, NS, 8, N_PAD), jnp.float32),
        mesh=_mesh,
        compiler_params=pltpu.CompilerParams(needs_layout_passes=False),
        scratch_types=[pltpu.VMEM((CH2,), jnp.int32),
                       pltpu.VMEM((8, CH2), jnp.float32),
                       pltpu.VMEM((8, N_PAD), jnp.float32)],
    )
    return f(objc, m_in)


def _sc_c1_body(subc, objc, s1, s2, scores_out, stats_out,
                sub_v, obj_v, s1_v, s2_v, sc_v, st_v):
    w = _wid()
    iot = lax.iota(jnp.int32, L)
    nb = _nblocks(w)
    m0 = jnp.full((L,), NEG, jnp.float32)
    s0 = jnp.zeros((L,), jnp.float32)

    def blk(j, MS):
        b = w + j * NW
        base = b * EB
        pltpu.sync_copy(subc.at[pl.ds(base, EB)], sub_v)
        pltpu.sync_copy(objc.at[pl.ds(base, EB)], obj_v)
        pltpu.sync_copy(s1.at[sub_v], s1_v)
        pltpu.sync_copy(s2.at[obj_v], s2_v)

        def grp(g, MS2):
            M, S = MS2
            x = s1_v[pl.ds(g * L, L)] + s2_v[pl.ds(g * L, L)]
            x = jnp.maximum(x, 0.2 * x)
            sc_v[pl.ds(g * L, L)] = x
            mn = jnp.maximum(M, x)
            S = S * jnp.exp(M - mn) + jnp.exp(x - mn)
            return (mn, S)

        MS = lax.fori_loop(0, EB // L, grp, MS)
        pltpu.sync_copy(sc_v, scores_out.at[pl.ds(base, EB)])
        return MS

    M, S = lax.fori_loop(0, nb, blk, (m0, s0))
    plsc.store_scatter(st_v, [_splat(0), iot], M)
    plsc.store_scatter(st_v, [_splat(1), iot], S)
    pltpu.sync_copy(st_v, stats_out.at[w])


def _sc_c1(subc, objc, s1, s2):
    f = pl.kernel(
        _sc_c1_body,
        out_type=[jax.ShapeDtypeStruct((E,), jnp.float32),
                  jax.ShapeDtypeStruct((NW, 2, L), jnp.float32)],
        mesh=_mesh,
        compiler_params=pltpu.CompilerParams(needs_layout_passes=False),
        scratch_types=[pltpu.VMEM((EB,), jnp.int32),
                       pltpu.VMEM((EB,), jnp.int32),
                       pltpu.VMEM((EB,), jnp.float32),
                       pltpu.VMEM((EB,), jnp.float32),
                       pltpu.VMEM((EB,), jnp.float32),
                       pltpu.VMEM((2, L), jnp.float32)],
    )
    return f(subc, objc, s1, s2)


def _sc_c2_body(subc, objc, scores, stats, h2, out_p,
                sub_v, obj_v, sc_v, w_v, h2r_v, z_v, st_v, osh):
    w = _wid()
    sid = lax.axis_index("s")
    cid = lax.axis_index("c")
    iot = lax.iota(jnp.int32, L)
    pltpu.sync_copy(stats, st_v)

    def comb(t, MS):
        M, S = MS
        mv = plsc.load_gather(st_v, [_splat(t), _splat(0), iot])
        sv = plsc.load_gather(st_v, [_splat(t), _splat(1), iot])
        mn = jnp.maximum(M, mv)
        S = S * jnp.exp(M - mn) + sv * jnp.exp(mv - mn)
        return (mn, S)

    M, S = lax.fori_loop(0, NW, comb,
                         (jnp.full((L,), NEG, jnp.float32),
                          jnp.zeros((L,), jnp.float32)))
    ms = lax.reduce_max(M, (0,))
    zs = lax.reduce_sum(S * jnp.exp(M - ms), (0,))
    inv = jnp.ones((L,), jnp.float32) / jnp.broadcast_to(zs, (L,))

    def z(i, c):
        for k in range(8):
            z_v[i, pl.ds(k * L, L)] = jnp.zeros((L,), jnp.float32)
        return c

    lax.fori_loop(0, EB, z, 0)
    for r in range(5):
        pltpu.sync_copy(z_v, osh.at[pl.ds(sid * 640 + r * EB, EB), :])
    plsc.subcore_barrier()

    nb = _nblocks(w)

    def blk(j, carry):
        b = w + j * NW
        base = b * EB
        pltpu.sync_copy(subc.at[pl.ds(base, EB)], sub_v)
        pltpu.sync_copy(objc.at[pl.ds(base, EB)], obj_v)
        pltpu.sync_copy(scores.at[pl.ds(base, EB)], sc_v)
        pltpu.sync_copy(h2.at[obj_v], h2r_v)

        def grp(g, c):
            x = sc_v[pl.ds(g * L, L)]
            w_v[pl.ds(g * L, L)] = jnp.exp(x - ms) * inv
            return c

        lax.fori_loop(0, EB // L, grp, 0)

        def scale(i, c):
            wv = plsc.load_gather(w_v, [_splat(i)])
            for k in range(8):
                sl = pl.ds(k * L, L)
                h2r_v[i, sl] = h2r_v[i, sl] * wv
            return c

        lax.fori_loop(0, EB, scale, 0)
        pltpu.sync_copy(h2r_v, osh.at[sub_v], add=True)
        return carry

    lax.fori_loop(0, nb, blk, 0)
    plsc.subcore_barrier()
    for r in range(5):
        pltpu.sync_copy(osh.at[pl.ds(sid * 640 + r * EB, EB), :],
                        out_p.at[cid, pl.ds(sid * 640 + r * EB, EB), :])


def _sc_c2(subc, objc, scores, stats, h2):
    f = pl.kernel(
        _sc_c2_body,
        out_type=jax.ShapeDtypeStruct((NC, N_PAD, 128), jnp.float32),
        mesh=_mesh,
        compiler_params=pltpu.CompilerParams(needs_layout_passes=False),
        scratch_types=[pltpu.VMEM((EB,), jnp.int32),
                       pltpu.VMEM((EB,), jnp.int32),
                       pltpu.VMEM((EB,), jnp.float32),
                       pltpu.VMEM((EB,), jnp.float32),
                       pltpu.VMEM((EB, 128), jnp.float32),
                       pltpu.VMEM((EB, 128), jnp.float32),
                       pltpu.VMEM((NW, 2, L), jnp.float32),
                       pltpu.VMEM_SHARED((N_PAD, 128), jnp.float32)],
    )
    return f(subc, objc, scores, stats, h2)


def kernel(q_sub, q_rel, hidden, edges, nodes, old_nodes_new_idx, batchsize,
           rela_embed, Ws_attn, Wr_attn, Wqr_attn, bqr, w_alpha, b_alpha, W_h,
           attn_fc_w, attn_fc_b, W_node_w, W_node_b):
    rp = jnp.pad(rela_embed, ((0, N_PAD - rela_embed.shape[0]), (0, 0)))
    edges = edges.astype(jnp.int32)
    q_rel = q_rel.astype(jnp.int32)

    a_s = _tc0a(hidden, Ws_attn)
    a_r, a_q, nrel = _tc0b(rp, Wr_attn, Wqr_attn, bqr)
    T, U, subc, objc = _sc_a(edges.reshape(-1), q_rel, a_s, a_r, a_q, hidden, nrel)
    M = _tc1(T, U, w_alpha, b_alpha)
    agg = _sc_b(objc, M)
    s1, s2, h2 = _tc2(agg, W_h, attn_fc_w[:128], attn_fc_w[128:],
                      attn_fc_b, W_node_w, W_node_b)
    scores, stats = _sc_c1(subc, objc, s1, s2)
    parts = _sc_c2(subc, objc, scores, stats, h2)
    out = _tc3(parts)
    return out[:10000]
